# Initial kernel scaffold; baseline (speedup 1.0000x reference)
import jax
import jax.numpy as jnp
from jax.experimental import pallas as pl


def kernel(x, edge_index, edge_attr, emb, mlp_w1, mlp_b1, mlp_w2, mlp_b2, cf1_w, cf2_w, cf2_b, lin_w, lin_b, lin1_w, lin1_b, lin2_w, lin2_b, ro_w, ro_b):
    raise NotImplementedError("write your pallas kernel here")


# speedup vs baseline: 1.9545x; 1.9545x over previous
import functools

import jax
import jax.numpy as jnp
from jax import lax
from jax.experimental import pallas as pl
from jax.experimental.pallas import tpu as pltpu
from jax.experimental.pallas import tpu_sc as plsc

N = 10000
E = 320000
HC = 128
NF = 128
NG = 50
NI = 6
CUT = 10.0
OUT = 12
H2 = HC // 2

NP = 10240
EP = 327680
NWK = 32
KE = 128
CHUNKS = EP // NWK // KE
STRIPE = NP // 16

BN = 1024
BE = 2048

_LOG2 = 0.6931471805599453


def _ssp(x):
    return jax.nn.softplus(x) - _LOG2


def _embed_body(xb, emb, cf1, h_out, hx_out):
    cls = lax.broadcasted_iota(jnp.int32, (BN, 100), 1)
    oh = (xb[...] == cls).astype(jnp.float32)
    h0 = jnp.dot(oh, emb[...], preferred_element_type=jnp.float32)
    h_out[...] = h0
    hx_out[...] = jnp.dot(h0, cf1[...], preferred_element_type=jnp.float32)


def _embed(xp, emb, cf1):
    return pl.pallas_call(
        _embed_body,
        grid=(NP // BN,),
        in_specs=[
            pl.BlockSpec((BN, 1), lambda n: (n, 0)),
            pl.BlockSpec((100, HC), lambda n: (0, 0)),
            pl.BlockSpec((HC, HC), lambda n: (0, 0)),
        ],
        out_specs=[
            pl.BlockSpec((BN, HC), lambda n: (n, 0)),
            pl.BlockSpec((BN, HC), lambda n: (n, 0)),
        ],
        out_shape=[
            jax.ShapeDtypeStruct((NP, HC), jnp.float32),
            jax.ShapeDtypeStruct((NP, HC), jnp.float32),
        ],
    )(xp, emb, cf1)


def _filter_body(ea, w1, b1, w2, b2, wf_out):
    a = ea[...]
    step = CUT / (NG - 1)
    off = step * lax.broadcasted_iota(jnp.int32, (1, NG), 1).astype(jnp.float32)
    coeff = -0.5 / (step * step)
    ed = jnp.exp(coeff * (a - off) ** 2)
    z = jnp.dot(ed, w1[...], preferred_element_type=jnp.float32) + b1[...]
    z = _ssp(z)
    z = jnp.dot(z, w2[...], preferred_element_type=jnp.float32) + b2[...]
    cc = 0.5 * (jnp.cos(a * (jnp.pi / CUT)) + 1.0)
    wf_out[...] = z * cc


def _filter(ea2, w1, b1, w2, b2):
    return pl.pallas_call(
        _filter_body,
        grid=(EP // BE,),
        in_specs=[
            pl.BlockSpec((BE, 1), lambda n: (n, 0)),
            pl.BlockSpec((NG, NF), lambda n: (0, 0)),
            pl.BlockSpec((1, NF), lambda n: (0, 0)),
            pl.BlockSpec((NF, NF), lambda n: (0, 0)),
            pl.BlockSpec((1, NF), lambda n: (0, 0)),
        ],
        out_specs=pl.BlockSpec((BE, NF), lambda n: (n, 0)),
        out_shape=jax.ShapeDtypeStruct((EP, NF), jnp.float32),
    )(ea2, w1, b1, w2, b2)


def _cfconv_body(hx_hbm, wf_hbm, src_hbm, dst_hbm, zer_hbm, out_hbm,
                 sidx, didx, wfv, rows, agg, sem):
    c = lax.axis_index("c")
    s = lax.axis_index("s")
    pltpu.sync_copy(zer_hbm, agg.at[pl.ds(s * STRIPE, STRIPE)])
    plsc.subcore_barrier()
    w = s * 2 + c
    base = w * (EP // NWK)

    def chunk(g, carry):
        ofs = base + g * KE
        pltpu.sync_copy(src_hbm.at[pl.ds(ofs, KE)], sidx)
        pltpu.sync_copy(dst_hbm.at[pl.ds(ofs, KE)], didx)
        pltpu.sync_copy(wf_hbm.at[pl.ds(ofs, KE), :], wfv)
        pltpu.async_copy(hx_hbm.at[sidx], rows, sem).wait()

        def mul(j, cc):
            for v in range(8):
                sl = pl.ds(v * 16, 16)
                rows[j, sl] = rows[j, sl] * wfv[j, sl]
            return cc

        lax.fori_loop(0, KE, mul, 0)
        pltpu.sync_copy(rows, agg.at[didx], add=True)
        return carry

    lax.fori_loop(0, CHUNKS, chunk, 0)
    plsc.subcore_barrier()
    stripe = pl.ds(s * STRIPE, STRIPE)
    pltpu.sync_copy(agg.at[stripe], out_hbm.at[c, stripe, :])


def _cfconv(hx, wf, srcp, dstp, zer):
    mesh = plsc.VectorSubcoreMesh(core_axis_name="c", subcore_axis_name="s")
    fn = functools.partial(
        pl.kernel,
        mesh=mesh,
        out_type=jax.ShapeDtypeStruct((2, NP, HC), jnp.float32),
        scratch_types=[
            pltpu.VMEM((KE,), jnp.int32),
            pltpu.VMEM((KE,), jnp.int32),
            pltpu.VMEM((KE, HC), jnp.float32),
            pltpu.VMEM((KE, HC), jnp.float32),
            pltpu.VMEM_SHARED((NP, HC), jnp.float32),
            pltpu.SemaphoreType.DMA,
        ],
    )(_cfconv_body)
    return fn(hx, wf, srcp, dstp, zer)


def _update_body(aggp, h, cf2w, cf2b, linw, linb, cf1n, h_out, hx_out):
    agg = aggp[0] + aggp[1]
    t = _ssp(jnp.dot(agg, cf2w[...], preferred_element_type=jnp.float32)
             + cf2b[...])
    hc = jnp.dot(t, linw[...], preferred_element_type=jnp.float32) + linb[...]
    hn = h[...] + hc
    h_out[...] = hn
    hx_out[...] = jnp.dot(hn, cf1n[...], preferred_element_type=jnp.float32)


def _update(aggp, h, cf2w, cf2b, linw, linb, cf1n):
    return pl.pallas_call(
        _update_body,
        grid=(NP // BN,),
        in_specs=[
            pl.BlockSpec((2, BN, HC), lambda n: (0, n, 0)),
            pl.BlockSpec((BN, HC), lambda n: (n, 0)),
            pl.BlockSpec((HC, HC), lambda n: (0, 0)),
            pl.BlockSpec((1, HC), lambda n: (0, 0)),
            pl.BlockSpec((HC, HC), lambda n: (0, 0)),
            pl.BlockSpec((1, HC), lambda n: (0, 0)),
            pl.BlockSpec((HC, HC), lambda n: (0, 0)),
        ],
        out_specs=[
            pl.BlockSpec((BN, HC), lambda n: (n, 0)),
            pl.BlockSpec((BN, HC), lambda n: (n, 0)),
        ],
        out_shape=[
            jax.ShapeDtypeStruct((NP, HC), jnp.float32),
            jax.ShapeDtypeStruct((NP, HC), jnp.float32),
        ],
    )(aggp, h, cf2w, cf2b, linw, linb, cf1n)


def _readout_body(h, l1w, l1b, l2w, l2b, row, rob, out, acc):
    pid = pl.program_id(0)
    npr = pl.num_programs(0)

    @pl.when(pid == 0)
    def _init():
        acc[...] = jnp.zeros_like(acc)

    gid = pid * BN + lax.broadcasted_iota(jnp.int32, (BN, 1), 0)
    mask = (gid < N).astype(jnp.float32)
    t = _ssp(jnp.dot(h[...], l1w[...], preferred_element_type=jnp.float32)
             + l1b[...])
    acc[...] += jnp.sum(t * mask, axis=0, keepdims=True)

    @pl.when(pid == npr - 1)
    def _fin():
        s = jnp.dot(acc[...], l2w[...],
                    preferred_element_type=jnp.float32) + float(N) * l2b[...]
        out[...] = jnp.dot(s, row[...],
                           preferred_element_type=jnp.float32) + rob[...]


def _readout(h, l1w, l1b, l2w, l2b, row, rob):
    return pl.pallas_call(
        _readout_body,
        grid=(NP // BN,),
        in_specs=[
            pl.BlockSpec((BN, HC), lambda n: (n, 0)),
            pl.BlockSpec((HC, H2), lambda n: (0, 0)),
            pl.BlockSpec((1, H2), lambda n: (0, 0)),
            pl.BlockSpec((H2, H2), lambda n: (0, 0)),
            pl.BlockSpec((1, H2), lambda n: (0, 0)),
            pl.BlockSpec((H2, OUT), lambda n: (0, 0)),
            pl.BlockSpec((1, OUT), lambda n: (0, 0)),
        ],
        out_specs=pl.BlockSpec((1, OUT), lambda n: (0, 0)),
        out_shape=jax.ShapeDtypeStruct((1, OUT), jnp.float32),
        scratch_shapes=[pltpu.VMEM((1, H2), jnp.float32)],
    )(h, l1w, l1b, l2w, l2b, row, rob)


def kernel(x, edge_index, edge_attr, emb, mlp_w1, mlp_b1, mlp_w2, mlp_b2,
           cf1_w, cf2_w, cf2_b, lin_w, lin_b, lin1_w, lin1_b,
           lin2_w, lin2_b, ro_w, ro_b):
    pad_e = EP - E
    fill = (jnp.arange(pad_e, dtype=jnp.int32) * 37) % N
    srcp = jnp.concatenate([edge_index[0].astype(jnp.int32), fill])
    dstp = jnp.concatenate([edge_index[1].astype(jnp.int32), fill])
    eap = jnp.concatenate(
        [edge_attr.astype(jnp.float32),
         jnp.full((pad_e,), CUT, jnp.float32)]).reshape(EP, 1)
    xp = jnp.pad(x.astype(jnp.int32), (0, NP - N)).reshape(NP, 1)
    zer = jnp.zeros((STRIPE, HC), jnp.float32)

    b1 = mlp_b1.reshape(NI, 1, NF)
    b2 = mlp_b2.reshape(NI, 1, NF)
    c2b = cf2_b.reshape(NI, 1, HC)
    lnb = lin_b.reshape(NI, 1, HC)

    h, hx = _embed(xp, emb.astype(jnp.float32), cf1_w[0])
    for i in range(NI):
        wf = _filter(eap, mlp_w1[i], b1[i], mlp_w2[i], b2[i])
        aggp = _cfconv(hx, wf, srcp, dstp, zer)
        h, hx = _update(aggp, h, cf2_w[i], c2b[i], lin_w[i], lnb[i],
                        cf1_w[(i + 1) % NI])
    return _readout(h, lin1_w, lin1_b.reshape(1, H2), lin2_w,
                    lin2_b.reshape(1, H2), ro_w, ro_b.reshape(1, OUT))
